# baseline (speedup 1.0000x reference)
import functools

import jax
import jax.numpy as jnp
from jax import lax
from jax.experimental import pallas as pl
from jax.experimental.pallas import tpu as pltpu
from jax.experimental.pallas import tpu_sc as plsc

N = 10000
E = 320000
D = 128

NC = 2
NS = 16
L = 16
NW = NC * NS
EPW = 10240
EP = NW * EPW
C = 16
NCH = EPW // C
NB = 128
NBLK = NCH // NB
NB2 = NB // 2
NK = 4
NOP = NB2 // NK
NP = 10240
RPT = NP // NS
ZR = 16

_BCAST_DNUMS = lax.GatherDimensionNumbers(
    offset_dims=(), collapsed_slice_dims=(0,), start_index_map=(0,))


def _lane_bcast(v16, lane):
    idx = jnp.full((L, 1), lane, dtype=jnp.int32)
    return lax.gather(v16, idx, _BCAST_DNUMS, slice_sizes=(1,),
                      mode=lax.GatherScatterMode.PROMISE_IN_BOUNDS)


def _mm_body(x_ref, w_ref, o_ref):
    o_ref[...] = jnp.dot(x_ref[...], w_ref[...],
                         preferred_element_type=jnp.float32)


def _combine_body(p_ref, o_ref):
    o_ref[...] = jnp.maximum(p_ref[0] + p_ref[1], 0.0)


def _sc_body(h_hbm, src_hbm, dst_hbm, vals_hbm, out_hbm,
             src_v, dst_v, vals_v,
             g0, g1, g2, g3, zb, acc_sh,
             ml0, ml1, ml2, ml3, mh0, mh1, mh2, mh3,
             ms0, ms1, ms2, ms3, mz):
    cid = lax.axis_index("c")
    sid = lax.axis_index("s")
    wid = cid * NS + sid
    gbufs = (g0, g1, g2, g3)
    losems = (ml0, ml1, ml2, ml3)
    hisems = (mh0, mh1, mh2, mh3)
    ssems = (ms0, ms1, ms2, ms3)

    def _gather(p, k):
        pltpu.async_copy(h_hbm.at[src_v.at[2 * p]],
                         gbufs[k].at[pl.ds(0, C)], losems[k])
        pltpu.async_copy(h_hbm.at[src_v.at[2 * p + 1]],
                         gbufs[k].at[pl.ds(C, C)], hisems[k])

    def _gwait(k):
        pltpu.make_async_copy(h_hbm.at[src_v.at[0]],
                              gbufs[k].at[pl.ds(0, C)], losems[k]).wait()
        pltpu.make_async_copy(h_hbm.at[src_v.at[0]],
                              gbufs[k].at[pl.ds(C, C)], hisems[k]).wait()

    def _sfire(p, k):
        pltpu.async_copy(gbufs[k], acc_sh.at[dst_v.at[p]], ssems[k],
                         add=True)

    def _swait(k):
        pltpu.make_async_copy(gbufs[k], acc_sh.at[dst_v.at[0]],
                              ssems[k]).wait()

    def _scale(p, k):
        gb = gbufs[k]
        for h in range(2):
            v16 = vals_v[2 * p + h, :]

            def _e(lane, c2):
                v = _lane_bcast(v16, lane)
                e = h * C + lane
                for t in range(D // L):
                    gb[e, pl.ds(t * L, L)] = gb[e, pl.ds(t * L, L)] * v
                return c2
            lax.fori_loop(0, C, _e, 0)

    for blk in range(NBLK):
        pltpu.sync_copy(src_hbm.at[wid, pl.ds(blk * NB, NB)], src_v)
        pltpu.sync_copy(dst_hbm.at[wid, pl.ds(blk * NB2, NB2)], dst_v)
        pltpu.sync_copy(vals_hbm.at[wid, pl.ds(blk * NB, NB)], vals_v)

        for k in range(NK):
            _gather(k, k)

        if blk == 0:
            def _zrow(i, carry):
                for t in range(D // L):
                    zb[i, pl.ds(t * L, L)] = jnp.zeros((L,), jnp.float32)
                return carry
            lax.fori_loop(0, ZR, _zrow, 0)
            for b in range(RPT // ZR):
                pltpu.async_copy(
                    zb, acc_sh.at[pl.ds(sid * RPT + b * ZR, ZR)], mz)
            for b in range(RPT // ZR):
                pltpu.make_async_copy(
                    zb, acc_sh.at[pl.ds(sid * RPT, ZR)], mz).wait()
            plsc.subcore_barrier()

        def _oct(p4, carry):
            p0 = NK * p4
            for k in range(NK):
                p = p0 + k
                _gwait(k)
                _scale(p, k)
                _sfire(p, k)
                if k >= 2:
                    kk = k - 2

                    @pl.when(p4 < NOP - 1)
                    def _():
                        _swait(kk)
                        _gather(p0 + NK + kk, kk)

            @pl.when(p4 < NOP - 1)
            def _():
                for kk in (NK - 2, NK - 1):
                    _swait(kk)
                    _gather(p0 + NK + kk, kk)
            return carry
        lax.fori_loop(0, NOP, _oct, 0)
        for k in range(NK):
            _swait(k)

    plsc.subcore_barrier()
    pltpu.sync_copy(acc_sh.at[pl.ds(sid * RPT, RPT)],
                    out_hbm.at[cid, pl.ds(sid * RPT, RPT)])


_sc_call = functools.partial(
    pl.kernel,
    out_type=jax.ShapeDtypeStruct((NC, NP, D), jnp.float32),
    mesh=plsc.VectorSubcoreMesh(core_axis_name="c", subcore_axis_name="s"),
    compiler_params=pltpu.CompilerParams(use_tc_tiling_on_sc=False),
    scratch_types=(
        [pltpu.VMEM((NB, C), jnp.int32),
         pltpu.VMEM((NB2, 2 * C), jnp.int32),
         pltpu.VMEM((NB, C), jnp.float32)]
        + [pltpu.VMEM((2 * C, D), jnp.float32) for _ in range(NK)]
        + [pltpu.VMEM((ZR, D), jnp.float32)]
        + [pltpu.VMEM_SHARED((NP, D), jnp.float32)]
        + [pltpu.SemaphoreType.DMA for _ in range(3 * NK + 1)]
    ),
)(_sc_body)


def kernel(x, W, edge_index, adj_vals):
    h = pl.pallas_call(
        _mm_body,
        grid=(5,),
        in_specs=[pl.BlockSpec((N // 5, D), lambda i: (i, 0)),
                  pl.BlockSpec((D, D), lambda i: (0, 0))],
        out_specs=pl.BlockSpec((N // 5, D), lambda i: (i, 0)),
        out_shape=jax.ShapeDtypeStruct((N, D), jnp.float32),
    )(x, W)

    pad = EP - E
    pi = jnp.arange(pad, dtype=jnp.int32)
    src_r = jnp.concatenate([edge_index[1], pi % N]).reshape(NW, NCH, C)
    dst_r = jnp.concatenate(
        [edge_index[0], N + pi % (NP - N)]).reshape(NW, NCH // 2, 2 * C)
    vals_r = jnp.concatenate(
        [adj_vals, jnp.zeros((pad,), jnp.float32)]).reshape(NW, NCH, C)

    partials = _sc_call(h, src_r, dst_r, vals_r)

    out = pl.pallas_call(
        _combine_body,
        grid=(5,),
        in_specs=[pl.BlockSpec((NC, N // 5, D), lambda i: (0, i, 0))],
        out_specs=pl.BlockSpec((N // 5, D), lambda i: (i, 0)),
        out_shape=jax.ShapeDtypeStruct((N, D), jnp.float32),
    )(partials)
    return out


# speedup vs baseline: 1.1130x; 1.1130x over previous
import functools

import jax
import jax.numpy as jnp
from jax import lax
from jax.experimental import pallas as pl
from jax.experimental.pallas import tpu as pltpu
from jax.experimental.pallas import tpu_sc as plsc

N = 10000
E = 320000
D = 128

NC = 2
NS = 16
L = 16
NW = NC * NS
EPW = 10240
EP = NW * EPW
C = 16
NCH = EPW // C
NB = 128
NBLK = NCH // NB
NG = 8
NO = NB // NG
NP = 10240
RPT = NP // NS
ZR = 16

_BCAST_DNUMS = lax.GatherDimensionNumbers(
    offset_dims=(), collapsed_slice_dims=(0,), start_index_map=(0,))


def _lane_bcast(v16, lane):
    idx = jnp.full((L, 1), lane, dtype=jnp.int32)
    return lax.gather(v16, idx, _BCAST_DNUMS, slice_sizes=(1,),
                      mode=lax.GatherScatterMode.PROMISE_IN_BOUNDS)


def _mm_body(x_ref, w_ref, o_ref):
    o_ref[...] = jnp.dot(x_ref[...], w_ref[...],
                         preferred_element_type=jnp.float32)


def _combine_body(p_ref, o_ref):
    o_ref[...] = jnp.maximum(p_ref[0] + p_ref[1], 0.0)


def _sc_body(h_hbm, src_hbm, dst_hbm, vals_hbm, out_hbm,
             src_v, dst_v, vals_v,
             g0, g1, g2, g3, g4, g5, g6, g7, zb, acc_sh,
             mg0, mg1, mg2, mg3, mg4, mg5, mg6, mg7,
             ms0, ms1, ms2, ms3, ms4, ms5, ms6, ms7, mz):
    cid = lax.axis_index("c")
    sid = lax.axis_index("s")
    wid = cid * NS + sid
    gbufs = (g0, g1, g2, g3, g4, g5, g6, g7)
    gsems = (mg0, mg1, mg2, mg3, mg4, mg5, mg6, mg7)
    ssems = (ms0, ms1, ms2, ms3, ms4, ms5, ms6, ms7)

    def _gather(j, q):
        pltpu.async_copy(h_hbm.at[src_v.at[j]], gbufs[q], gsems[q])

    def _gwait(q):
        pltpu.make_async_copy(h_hbm.at[src_v.at[0]], gbufs[q],
                              gsems[q]).wait()

    def _sfire(j, q):
        pltpu.async_copy(gbufs[q], acc_sh.at[dst_v.at[j]], ssems[q],
                         add=True)

    def _swait(q):
        pltpu.make_async_copy(gbufs[q], acc_sh.at[dst_v.at[0]],
                              ssems[q]).wait()

    def _scale(j, q):
        gb = gbufs[q]
        v16 = vals_v[j, :]

        def _e(lane, c2):
            v = _lane_bcast(v16, lane)
            for t in range(D // L):
                gb[lane, pl.ds(t * L, L)] = gb[lane, pl.ds(t * L, L)] * v
            return c2
        lax.fori_loop(0, C, _e, 0)

    for blk in range(NBLK):
        pltpu.sync_copy(src_hbm.at[wid, pl.ds(blk * NB, NB)], src_v)
        pltpu.sync_copy(dst_hbm.at[wid, pl.ds(blk * NB, NB)], dst_v)
        pltpu.sync_copy(vals_hbm.at[wid, pl.ds(blk * NB, NB)], vals_v)

        for q in range(NG):
            _gather(q, q)

        if blk == 0:
            def _zrow(i, carry):
                for t in range(D // L):
                    zb[i, pl.ds(t * L, L)] = jnp.zeros((L,), jnp.float32)
                return carry
            lax.fori_loop(0, ZR, _zrow, 0)
            for b in range(RPT // ZR):
                pltpu.async_copy(
                    zb, acc_sh.at[pl.ds(sid * RPT + b * ZR, ZR)], mz)
            for b in range(RPT // ZR):
                pltpu.make_async_copy(
                    zb, acc_sh.at[pl.ds(sid * RPT, ZR)], mz).wait()
            plsc.subcore_barrier()

        def _oct(p, carry):
            j0 = NG * p
            for q in range(NG):
                j = j0 + q
                _gwait(q)
                _scale(j, q)
                _sfire(j, q)
                if q >= 2:
                    qq = q - 2

                    @pl.when(p < NO - 1)
                    def _():
                        _swait(qq)
                        _gather(j0 + NG + qq, qq)

            @pl.when(p < NO - 1)
            def _():
                for qq in (NG - 2, NG - 1):
                    _swait(qq)
                    _gather(j0 + NG + qq, qq)
            return carry
        lax.fori_loop(0, NO, _oct, 0)
        for q in range(NG):
            _swait(q)

    plsc.subcore_barrier()
    pltpu.sync_copy(acc_sh.at[pl.ds(sid * RPT, RPT)],
                    out_hbm.at[cid, pl.ds(sid * RPT, RPT)])


_sc_call = functools.partial(
    pl.kernel,
    out_type=jax.ShapeDtypeStruct((NC, NP, D), jnp.float32),
    mesh=plsc.VectorSubcoreMesh(core_axis_name="c", subcore_axis_name="s"),
    compiler_params=pltpu.CompilerParams(use_tc_tiling_on_sc=False),
    scratch_types=(
        [pltpu.VMEM((NB, C), jnp.int32),
         pltpu.VMEM((NB, C), jnp.int32),
         pltpu.VMEM((NB, C), jnp.float32)]
        + [pltpu.VMEM((C, D), jnp.float32) for _ in range(NG)]
        + [pltpu.VMEM((ZR, D), jnp.float32)]
        + [pltpu.VMEM_SHARED((NP, D), jnp.float32)]
        + [pltpu.SemaphoreType.DMA for _ in range(2 * NG + 1)]
    ),
)(_sc_body)


def kernel(x, W, edge_index, adj_vals):
    h = pl.pallas_call(
        _mm_body,
        grid=(5,),
        in_specs=[pl.BlockSpec((N // 5, D), lambda i: (i, 0)),
                  pl.BlockSpec((D, D), lambda i: (0, 0))],
        out_specs=pl.BlockSpec((N // 5, D), lambda i: (i, 0)),
        out_shape=jax.ShapeDtypeStruct((N, D), jnp.float32),
    )(x, W)

    pad = EP - E
    pi = jnp.arange(pad, dtype=jnp.int32)
    src_r = jnp.concatenate([edge_index[1], pi % N]).reshape(NW, NCH, C)
    dst_r = jnp.concatenate(
        [edge_index[0], N + pi % (NP - N)]).reshape(NW, NCH, C)
    vals_r = jnp.concatenate(
        [adj_vals, jnp.zeros((pad,), jnp.float32)]).reshape(NW, NCH, C)

    partials = _sc_call(h, src_r, dst_r, vals_r)

    out = pl.pallas_call(
        _combine_body,
        grid=(5,),
        in_specs=[pl.BlockSpec((NC, N // 5, D), lambda i: (0, i, 0))],
        out_specs=pl.BlockSpec((N // 5, D), lambda i: (i, 0)),
        out_shape=jax.ShapeDtypeStruct((N, D), jnp.float32),
    )(partials)
    return out


# speedup vs baseline: 1.1484x; 1.0318x over previous
import functools

import jax
import jax.numpy as jnp
from jax import lax
from jax.experimental import pallas as pl
from jax.experimental.pallas import tpu as pltpu
from jax.experimental.pallas import tpu_sc as plsc

N = 10000
E = 320000
D = 128

NC = 2
NS = 16
L = 16
NW = NC * NS
EPW = 10240
EP = NW * EPW
C = 16
NCH = EPW // C
NB = 160
NBLK = NCH // NB
NG = 10
NO = NB // NG
NP = 10240
RPT = NP // NS
ZR = 16

_BCAST_DNUMS = lax.GatherDimensionNumbers(
    offset_dims=(), collapsed_slice_dims=(0,), start_index_map=(0,))


def _lane_bcast(v16, lane):
    idx = jnp.full((L, 1), lane, dtype=jnp.int32)
    return lax.gather(v16, idx, _BCAST_DNUMS, slice_sizes=(1,),
                      mode=lax.GatherScatterMode.PROMISE_IN_BOUNDS)


def _mm_body(x_ref, w_ref, o_ref):
    o_ref[...] = jnp.dot(x_ref[...], w_ref[...],
                         preferred_element_type=jnp.float32)


def _combine_body(p_ref, o_ref):
    o_ref[...] = jnp.maximum(p_ref[0] + p_ref[1], 0.0)


def _sc_body(h_hbm, src_hbm, dst_hbm, vals_hbm, out_hbm,
             src_v, dst_v, vals_v,
             g0, g1, g2, g3, g4, g5, g6, g7, g8, g9, zb, acc_sh,
             mg0, mg1, mg2, mg3, mg4, mg5, mg6, mg7, mg8, mg9,
             ms0, ms1, ms2, ms3, ms4, ms5, ms6, ms7, ms8, ms9, mz):
    cid = lax.axis_index("c")
    sid = lax.axis_index("s")
    wid = cid * NS + sid
    gbufs = (g0, g1, g2, g3, g4, g5, g6, g7, g8, g9)
    gsems = (mg0, mg1, mg2, mg3, mg4, mg5, mg6, mg7, mg8, mg9)
    ssems = (ms0, ms1, ms2, ms3, ms4, ms5, ms6, ms7, ms8, ms9)

    def _gather(j, q):
        pltpu.async_copy(h_hbm.at[src_v.at[j]], gbufs[q], gsems[q])

    def _gwait(q):
        pltpu.make_async_copy(h_hbm.at[src_v.at[0]], gbufs[q],
                              gsems[q]).wait()

    def _sfire(j, q):
        pltpu.async_copy(gbufs[q], acc_sh.at[dst_v.at[j]], ssems[q],
                         add=True)

    def _swait(q):
        pltpu.make_async_copy(gbufs[q], acc_sh.at[dst_v.at[0]],
                              ssems[q]).wait()

    def _scale(j, q):
        gb = gbufs[q]
        v16 = vals_v[j, :]

        def _e(lane, c2):
            v = _lane_bcast(v16, lane)
            for t in range(D // L):
                gb[lane, pl.ds(t * L, L)] = gb[lane, pl.ds(t * L, L)] * v
            return c2
        lax.fori_loop(0, C, _e, 0)

    for blk in range(NBLK):
        pltpu.sync_copy(src_hbm.at[wid, pl.ds(blk * NB, NB)], src_v)
        pltpu.sync_copy(dst_hbm.at[wid, pl.ds(blk * NB, NB)], dst_v)
        pltpu.sync_copy(vals_hbm.at[wid, pl.ds(blk * NB, NB)], vals_v)

        for q in range(NG):
            _gather(q, q)

        if blk == 0:
            def _zrow(i, carry):
                for t in range(D // L):
                    zb[i, pl.ds(t * L, L)] = jnp.zeros((L,), jnp.float32)
                return carry
            lax.fori_loop(0, ZR, _zrow, 0)
            for b in range(RPT // ZR):
                pltpu.async_copy(
                    zb, acc_sh.at[pl.ds(sid * RPT + b * ZR, ZR)], mz)
            for b in range(RPT // ZR):
                pltpu.make_async_copy(
                    zb, acc_sh.at[pl.ds(sid * RPT, ZR)], mz).wait()
            plsc.subcore_barrier()

        def _oct(p, carry):
            j0 = NG * p
            for q in range(NG):
                j = j0 + q
                _gwait(q)
                _scale(j, q)
                _sfire(j, q)
                if q >= 2:
                    qq = q - 2

                    @pl.when(p < NO - 1)
                    def _():
                        _swait(qq)
                        _gather(j0 + NG + qq, qq)

            @pl.when(p < NO - 1)
            def _():
                for qq in (NG - 2, NG - 1):
                    _swait(qq)
                    _gather(j0 + NG + qq, qq)
            return carry
        lax.fori_loop(0, NO, _oct, 0)
        for q in range(NG):
            _swait(q)

    plsc.subcore_barrier()
    pltpu.sync_copy(acc_sh.at[pl.ds(sid * RPT, RPT)],
                    out_hbm.at[cid, pl.ds(sid * RPT, RPT)])


_sc_call = functools.partial(
    pl.kernel,
    out_type=jax.ShapeDtypeStruct((NC, NP, D), jnp.float32),
    mesh=plsc.VectorSubcoreMesh(core_axis_name="c", subcore_axis_name="s"),
    compiler_params=pltpu.CompilerParams(use_tc_tiling_on_sc=False),
    scratch_types=(
        [pltpu.VMEM((NB, C), jnp.int32),
         pltpu.VMEM((NB, C), jnp.int32),
         pltpu.VMEM((NB, C), jnp.float32)]
        + [pltpu.VMEM((C, D), jnp.float32) for _ in range(NG)]
        + [pltpu.VMEM((ZR, D), jnp.float32)]
        + [pltpu.VMEM_SHARED((NP, D), jnp.float32)]
        + [pltpu.SemaphoreType.DMA for _ in range(2 * NG + 1)]
    ),
)(_sc_body)


def kernel(x, W, edge_index, adj_vals):
    h = pl.pallas_call(
        _mm_body,
        grid=(5,),
        in_specs=[pl.BlockSpec((N // 5, D), lambda i: (i, 0)),
                  pl.BlockSpec((D, D), lambda i: (0, 0))],
        out_specs=pl.BlockSpec((N // 5, D), lambda i: (i, 0)),
        out_shape=jax.ShapeDtypeStruct((N, D), jnp.float32),
    )(x, W)

    pad = EP - E
    pi = jnp.arange(pad, dtype=jnp.int32)
    src_r = jnp.concatenate([edge_index[1], pi % N]).reshape(NW, NCH, C)
    dst_r = jnp.concatenate(
        [edge_index[0], N + pi % (NP - N)]).reshape(NW, NCH, C)
    vals_r = jnp.concatenate(
        [adj_vals, jnp.zeros((pad,), jnp.float32)]).reshape(NW, NCH, C)

    partials = _sc_call(h, src_r, dst_r, vals_r)

    out = pl.pallas_call(
        _combine_body,
        grid=(5,),
        in_specs=[pl.BlockSpec((NC, N // 5, D), lambda i: (0, i, 0))],
        out_specs=pl.BlockSpec((N // 5, D), lambda i: (i, 0)),
        out_shape=jax.ShapeDtypeStruct((N, D), jnp.float32),
    )(partials)
    return out


# speedup vs baseline: 1.1762x; 1.0243x over previous
import functools

import jax
import jax.numpy as jnp
from jax import lax
from jax.experimental import pallas as pl
from jax.experimental.pallas import tpu as pltpu
from jax.experimental.pallas import tpu_sc as plsc

N = 10000
E = 320000
D = 128

NC = 2
NS = 16
L = 16
NW = NC * NS
EPW = 10240
EP = NW * EPW
C = 16
NCH = EPW // C
NB = 80
NBLK = NCH // NB
NG = 10
NO = NB // NG
NP = 10240
RPT = NP // NS
ZR = 16

_BCAST_DNUMS = lax.GatherDimensionNumbers(
    offset_dims=(), collapsed_slice_dims=(0,), start_index_map=(0,))


def _lane_bcast(v16, lane):
    idx = jnp.full((L, 1), lane, dtype=jnp.int32)
    return lax.gather(v16, idx, _BCAST_DNUMS, slice_sizes=(1,),
                      mode=lax.GatherScatterMode.PROMISE_IN_BOUNDS)


def _mm_body(x_ref, w_ref, o_ref):
    o_ref[...] = jnp.dot(x_ref[...], w_ref[...],
                         preferred_element_type=jnp.float32)


def _combine_body(p_ref, o_ref):
    o_ref[...] = jnp.maximum(p_ref[0] + p_ref[1], 0.0)


def _sc_body(h_hbm, src_hbm, dst_hbm, vals_hbm, out_hbm,
             src_v, dst_v, vals_v,
             g0, g1, g2, g3, g4, g5, g6, g7, g8, g9, zb, acc_sh,
             mg0, mg1, mg2, mg3, mg4, mg5, mg6, mg7, mg8, mg9,
             ms0, ms1, ms2, ms3, ms4, ms5, ms6, ms7, ms8, ms9, mz, mst):
    cid = lax.axis_index("c")
    sid = lax.axis_index("s")
    wid = cid * NS + sid
    gbufs = (g0, g1, g2, g3, g4, g5, g6, g7, g8, g9)
    gsems = (mg0, mg1, mg2, mg3, mg4, mg5, mg6, mg7, mg8, mg9)
    ssems = (ms0, ms1, ms2, ms3, ms4, ms5, ms6, ms7, ms8, ms9)

    def _gather(bank, j, q):
        pltpu.async_copy(h_hbm.at[src_v.at[bank, j]], gbufs[q], gsems[q])

    def _gwait(q):
        pltpu.make_async_copy(h_hbm.at[src_v.at[0, 0]], gbufs[q],
                              gsems[q]).wait()

    def _sfire(bank, j, q):
        pltpu.async_copy(gbufs[q], acc_sh.at[dst_v.at[bank, j]], ssems[q],
                         add=True)

    def _swait(q):
        pltpu.make_async_copy(gbufs[q], acc_sh.at[dst_v.at[0, 0]],
                              ssems[q]).wait()

    def _scale(bank, j, q):
        gb = gbufs[q]
        v16 = vals_v[bank, j, :]

        def _e(lane, c2):
            v = _lane_bcast(v16, lane)
            for t in range(D // L):
                gb[lane, pl.ds(t * L, L)] = gb[lane, pl.ds(t * L, L)] * v
            return c2
        lax.fori_loop(0, C, _e, 0)

    def _stage(bank, blk, copy):
        copy(src_hbm.at[wid, pl.ds(blk * NB, NB)], src_v.at[bank])
        copy(dst_hbm.at[wid, pl.ds(blk * NB, NB)], dst_v.at[bank])
        copy(vals_hbm.at[wid, pl.ds(blk * NB, NB)], vals_v.at[bank])

    def _stage_async(bank, blk):
        _stage(bank, blk, lambda s, d: pltpu.async_copy(s, d, mst))

    def _stage_wait(bank):
        pltpu.make_async_copy(src_hbm.at[wid, pl.ds(0, NB)],
                              src_v.at[bank], mst).wait()
        pltpu.make_async_copy(dst_hbm.at[wid, pl.ds(0, NB)],
                              dst_v.at[bank], mst).wait()
        pltpu.make_async_copy(vals_hbm.at[wid, pl.ds(0, NB)],
                              vals_v.at[bank], mst).wait()

    _stage(0, 0, pltpu.sync_copy)
    for q in range(NG):
        _gather(0, q, q)

    def _zrow(i, carry):
        for t in range(D // L):
            zb[i, pl.ds(t * L, L)] = jnp.zeros((L,), jnp.float32)
        return carry
    lax.fori_loop(0, ZR, _zrow, 0)
    for b in range(RPT // ZR):
        pltpu.async_copy(zb, acc_sh.at[pl.ds(sid * RPT + b * ZR, ZR)], mz)
    for b in range(RPT // ZR):
        pltpu.make_async_copy(zb, acc_sh.at[pl.ds(sid * RPT, ZR)],
                              mz).wait()
    plsc.subcore_barrier()

    for blk in range(NBLK):
        bank = blk % 2
        nbank = 1 - bank
        if blk + 1 < NBLK:
            _stage_async(nbank, blk + 1)

        def _oct(p, carry):
            j0 = NG * p
            for q in range(NG):
                j = j0 + q
                _gwait(q)
                _scale(bank, j, q)
                _sfire(bank, j, q)
                if q >= 2:
                    _swait(q - 2)
                    _gather(bank, j0 + NG + q - 2, q - 2)
            for qq in (NG - 2, NG - 1):
                _swait(qq)
                _gather(bank, j0 + NG + qq, qq)
            return carry
        lax.fori_loop(0, NO - 1, _oct, 0)

        if blk + 1 < NBLK:
            _stage_wait(nbank)
        j0 = NG * (NO - 1)
        for q in range(NG):
            _gwait(q)
            _scale(bank, j0 + q, q)
            _sfire(bank, j0 + q, q)
            if blk + 1 < NBLK and q >= 2:
                _swait(q - 2)
                _gather(nbank, q - 2, q - 2)
        if blk + 1 < NBLK:
            for qq in (NG - 2, NG - 1):
                _swait(qq)
                _gather(nbank, qq, qq)

    for q in range(NG):
        _swait(q)

    plsc.subcore_barrier()
    pltpu.sync_copy(acc_sh.at[pl.ds(sid * RPT, RPT)],
                    out_hbm.at[cid, pl.ds(sid * RPT, RPT)])


_sc_call = functools.partial(
    pl.kernel,
    out_type=jax.ShapeDtypeStruct((NC, NP, D), jnp.float32),
    mesh=plsc.VectorSubcoreMesh(core_axis_name="c", subcore_axis_name="s"),
    compiler_params=pltpu.CompilerParams(use_tc_tiling_on_sc=False),
    scratch_types=(
        [pltpu.VMEM((2, NB, C), jnp.int32),
         pltpu.VMEM((2, NB, C), jnp.int32),
         pltpu.VMEM((2, NB, C), jnp.float32)]
        + [pltpu.VMEM((C, D), jnp.float32) for _ in range(NG)]
        + [pltpu.VMEM((ZR, D), jnp.float32)]
        + [pltpu.VMEM_SHARED((NP, D), jnp.float32)]
        + [pltpu.SemaphoreType.DMA for _ in range(2 * NG + 2)]
    ),
)(_sc_body)


def kernel(x, W, edge_index, adj_vals):
    h = pl.pallas_call(
        _mm_body,
        grid=(5,),
        in_specs=[pl.BlockSpec((N // 5, D), lambda i: (i, 0)),
                  pl.BlockSpec((D, D), lambda i: (0, 0))],
        out_specs=pl.BlockSpec((N // 5, D), lambda i: (i, 0)),
        out_shape=jax.ShapeDtypeStruct((N, D), jnp.float32),
    )(x, W)

    pad = EP - E
    pi = jnp.arange(pad, dtype=jnp.int32)
    src_r = jnp.concatenate([edge_index[1], pi % N]).reshape(NW, NCH, C)
    dst_r = jnp.concatenate(
        [edge_index[0], N + pi % (NP - N)]).reshape(NW, NCH, C)
    vals_r = jnp.concatenate(
        [adj_vals, jnp.zeros((pad,), jnp.float32)]).reshape(NW, NCH, C)

    partials = _sc_call(h, src_r, dst_r, vals_r)

    out = pl.pallas_call(
        _combine_body,
        grid=(5,),
        in_specs=[pl.BlockSpec((NC, N // 5, D), lambda i: (0, i, 0))],
        out_specs=pl.BlockSpec((N // 5, D), lambda i: (i, 0)),
        out_shape=jax.ShapeDtypeStruct((N, D), jnp.float32),
    )(partials)
    return out
